# baseline (speedup 1.0000x reference)
import functools

import jax
import jax.numpy as jnp
from jax import lax
from jax.experimental import pallas as pl
from jax.experimental.pallas import tpu as pltpu
from jax.experimental.pallas import tpu_sc as plsc

VOCAB = 100000
EMBED_DIM = 128
NUM_CORES = 2
NUM_SUBCORES = 16
NUM_WORKERS = NUM_CORES * NUM_SUBCORES
CHUNK = 128
NBUF = 4


def _embed_kernel(total, chunks_per_worker):
    mesh = plsc.VectorSubcoreMesh(core_axis_name="c", subcore_axis_name="s")
    n = chunks_per_worker

    @functools.partial(
        pl.kernel,
        out_type=jax.ShapeDtypeStruct((total, EMBED_DIM), jnp.float32),
        mesh=mesh,
        scratch_types=[
            pltpu.VMEM((n, CHUNK), jnp.int32),
            pltpu.VMEM((NBUF, CHUNK, EMBED_DIM), jnp.float32),
            pltpu.SemaphoreType.DMA((NBUF,)),
            pltpu.SemaphoreType.DMA((NBUF,)),
        ],
    )
    def k(idx_hbm, table_hbm, out_hbm, idx_v, rows_v, gsem, wsem):
        wid = lax.axis_index("s") * NUM_CORES + lax.axis_index("c")
        base = wid * (n * CHUNK)
        pltpu.sync_copy(idx_hbm.at[wid], idx_v)

        def gather(j, b):
            return pltpu.make_async_copy(
                table_hbm.at[idx_v.at[j]], rows_v.at[b], gsem.at[b]
            )

        def writeback(j, b):
            return pltpu.make_async_copy(
                rows_v.at[b], out_hbm.at[pl.ds(base + j * CHUNK, CHUNK)], wsem.at[b]
            )

        for jj in range(NBUF - 1):
            gather(jj, jj).start()

        @pl.loop(0, n, step=NBUF)
        def chunk_loop(j0):
            for b in range(NBUF):
                j = j0 + b
                gather(j, b).wait()
                writeback(j, b).start()
                nb = (b + NBUF - 1) % NBUF

                @pl.when(j + NBUF - 1 < n)
                def _start_next():
                    @pl.when(j >= 1)
                    def _free_buf():
                        writeback(j - 1, nb).wait()

                    gather(j + NBUF - 1, nb).start()

        for jj in range(n - NBUF, n):
            writeback(jj, jj % NBUF).wait()

    return k


def kernel(x, table):
    batch, hist = x.shape
    total = batch * hist
    chunks_per_worker = total // (NUM_WORKERS * CHUNK)
    idx = x.astype(jnp.int32).reshape(NUM_WORKERS, chunks_per_worker, CHUNK)
    out = _embed_kernel(total, chunks_per_worker)(idx, table)
    return out.reshape(batch, hist, EMBED_DIM)


# speedup vs baseline: 1.0042x; 1.0042x over previous
import functools

import jax
import jax.numpy as jnp
from jax import lax
from jax.experimental import pallas as pl
from jax.experimental.pallas import tpu as pltpu
from jax.experimental.pallas import tpu_sc as plsc

VOCAB = 100000
EMBED_DIM = 128
NUM_CORES = 2
NUM_SUBCORES = 16
NUM_WORKERS = NUM_CORES * NUM_SUBCORES
CHUNK = 128
GPM = 2
NBUF = 2


def _embed_kernel(total, chunks_per_worker):
    mesh = plsc.VectorSubcoreMesh(core_axis_name="c", subcore_axis_name="s")
    n = chunks_per_worker
    nm = n // GPM

    @functools.partial(
        pl.kernel,
        out_type=jax.ShapeDtypeStruct((total, EMBED_DIM), jnp.float32),
        mesh=mesh,
        scratch_types=[
            pltpu.VMEM((n, CHUNK), jnp.int32),
            pltpu.VMEM((NBUF, GPM * CHUNK, EMBED_DIM), jnp.float32),
            pltpu.SemaphoreType.DMA((NBUF,)),
            pltpu.SemaphoreType.DMA((NBUF,)),
        ],
    )
    def k(idx_hbm, table_hbm, out_hbm, idx_v, rows_v, gsem, wsem):
        wid = lax.axis_index("s") * NUM_CORES + lax.axis_index("c")
        base = wid * (n * CHUNK)
        pltpu.sync_copy(idx_hbm.at[wid], idx_v)

        def gather(m, b, h):
            return pltpu.make_async_copy(
                table_hbm.at[idx_v.at[m * GPM + h]],
                rows_v.at[b].at[pl.ds(h * CHUNK, CHUNK)],
                gsem.at[b],
            )

        def writeback(m, b):
            return pltpu.make_async_copy(
                rows_v.at[b],
                out_hbm.at[pl.ds(base + m * (GPM * CHUNK), GPM * CHUNK)],
                wsem.at[b],
            )

        for h in range(GPM):
            gather(0, 0, h).start()

        @pl.loop(0, nm, step=NBUF)
        def macro_loop(m0):
            for b in range(NBUF):
                m = m0 + b
                for h in range(GPM):
                    gather(m, b, h).wait()

                @pl.when(m >= 1)
                def _free_buf():
                    writeback(m - 1, 1 - b).wait()

                @pl.when(m + 1 < nm)
                def _start_next():
                    for h in range(GPM):
                        gather(m + 1, 1 - b, h).start()

                writeback(m, b).start()

        writeback(nm - 1, (nm - 1) % NBUF).wait()

    return k


def kernel(x, table):
    batch, hist = x.shape
    total = batch * hist
    chunks_per_worker = total // (NUM_WORKERS * CHUNK)
    idx = x.astype(jnp.int32).reshape(NUM_WORKERS, chunks_per_worker, CHUNK)
    out = _embed_kernel(total, chunks_per_worker)(idx, table)
    return out.reshape(batch, hist, EMBED_DIM)
